# Initial kernel scaffold; baseline (speedup 1.0000x reference)
import functools

import jax
import jax.numpy as jnp
from jax import lax
from jax.experimental import pallas as pl
from jax.experimental.pallas import tpu as pltpu
from jax.experimental.pallas import tpu_sc as plsc

B = 16384
L = 50
D = 128
NW = 32
BPW = B // NW
GRP = 64
VL = 16


def _accum_into(rows, stg, j, col0):
    def body(l, accs):
        return tuple(accs[c] + rows[l, pl.ds(c * VL, VL)] for c in range(D // VL))
    accs = tuple(rows[0, pl.ds(c * VL, VL)] for c in range(D // VL))
    accs = lax.fori_loop(1, L, body, accs)
    for c in range(D // VL):
        stg[j, pl.ds(col0 + c * VL, VL)] = accs[c]


_sc_mesh = plsc.VectorSubcoreMesh(core_axis_name="c", subcore_axis_name="s")


@functools.partial(
    pl.kernel,
    out_type=jax.ShapeDtypeStruct((B, 2 * D), jnp.float32),
    mesh=_sc_mesh,
    scratch_types=[
        pltpu.VMEM((BPW, L), jnp.int32),
        pltpu.VMEM((BPW, L), jnp.int32),
        pltpu.VMEM((L, D), jnp.float32),
        pltpu.VMEM((L, D), jnp.float32),
        pltpu.VMEM((GRP, 2 * D), jnp.float32),
        pltpu.SemaphoreType.DMA,
        pltpu.SemaphoreType.DMA,
    ],
)
def _sc_pool(xw_hbm, xb_hbm, table_hbm, out_hbm, idxw, idxb, rw, rb, ostg,
             sem_w, sem_b):
    wid = lax.axis_index("s") * 2 + lax.axis_index("c")
    base = wid * BPW
    pltpu.sync_copy(xw_hbm.at[pl.ds(base, BPW)], idxw)
    pltpu.sync_copy(xb_hbm.at[pl.ds(base, BPW)], idxb)

    def group_body(g, _):
        def row_body(i, _):
            r = g * GRP + i
            cpw = pltpu.async_copy(table_hbm.at[idxw.at[r]], rw, sem_w)
            cpb = pltpu.async_copy(table_hbm.at[idxb.at[r]], rb, sem_b)
            cpw.wait()
            _accum_into(rw, ostg, i, 0)
            cpb.wait()
            _accum_into(rb, ostg, i, D)
            return 0
        lax.fori_loop(0, GRP, row_body, 0)
        pltpu.sync_copy(ostg, out_hbm.at[pl.ds(base + g * GRP, GRP)])
        return 0

    lax.fori_loop(0, BPW // GRP, group_body, 0)


def _mlp_body(x_ref, w2_ref, b2_ref, w3_ref, b3_ref, w4_ref, b4_ref, o_ref):
    x = jnp.maximum(x_ref[:], 0.0)
    h = jnp.dot(x, w2_ref[:], preferred_element_type=jnp.float32) + b2_ref[:]
    h = jnp.maximum(h, 0.0)
    h = jnp.dot(h, w3_ref[:], preferred_element_type=jnp.float32) + b3_ref[:]
    h = jnp.maximum(h, 0.0)
    o_ref[:] = jnp.dot(h, w4_ref[:], preferred_element_type=jnp.float32) + b4_ref[:]


def _mlp(pooled, W2, b2, W3, b3, W4p, b4p):
    blk = 512
    return pl.pallas_call(
        _mlp_body,
        grid=(B // blk,),
        in_specs=[
            pl.BlockSpec((blk, 2 * D), lambda i: (i, 0)),
            pl.BlockSpec((2 * D, 32), lambda i: (0, 0)),
            pl.BlockSpec((1, 32), lambda i: (0, 0)),
            pl.BlockSpec((32, 32), lambda i: (0, 0)),
            pl.BlockSpec((1, 32), lambda i: (0, 0)),
            pl.BlockSpec((32, 128), lambda i: (0, 0)),
            pl.BlockSpec((1, 128), lambda i: (0, 0)),
        ],
        out_specs=pl.BlockSpec((blk, 128), lambda i: (i, 0)),
        out_shape=jax.ShapeDtypeStruct((B, 128), jnp.float32),
    )(pooled, W2, b2, W3, b3, W4p, b4p)


def kernel(x_w, x_b, table, W2, b2, W3, b3, W4, b4):
    pooled = _sc_pool(x_w.astype(jnp.int32), x_b.astype(jnp.int32), table)
    W4p = jnp.pad(W4, ((0, 0), (0, 127)))
    b4p = jnp.pad(b4.reshape(1, 1), ((0, 0), (0, 127)))
    out = _mlp(pooled, W2, b2.reshape(1, 32), W3, b3.reshape(1, 32), W4p, b4p)
    return out[:, :1]


# speedup vs baseline: 8.1353x; 8.1353x over previous
import functools

import jax
import jax.numpy as jnp
from jax import lax
from jax.experimental import pallas as pl
from jax.experimental.pallas import tpu as pltpu
from jax.experimental.pallas import tpu_sc as plsc

B = 16384
L = 50
D = 128
NW = 32
BPW = B // NW
GRP = 64
VL = 16


def _accum_into(rows, stg, j, col0):
    def body(l, accs):
        return tuple(accs[c] + rows[l, pl.ds(c * VL, VL)] for c in range(D // VL))
    accs = tuple(rows[0, pl.ds(c * VL, VL)] for c in range(D // VL))
    accs = lax.fori_loop(1, L, body, accs)
    for c in range(D // VL):
        stg[j, pl.ds(col0 + c * VL, VL)] = accs[c]


_sc_mesh = plsc.VectorSubcoreMesh(core_axis_name="c", subcore_axis_name="s")


@functools.partial(
    pl.kernel,
    out_type=jax.ShapeDtypeStruct((B, 2 * D), jnp.float32),
    mesh=_sc_mesh,
    scratch_types=[
        pltpu.VMEM((GRP, L), jnp.int32),
        pltpu.VMEM((GRP, L), jnp.int32),
        pltpu.VMEM((L, D), jnp.float32),
        pltpu.VMEM((L, D), jnp.float32),
        pltpu.VMEM((GRP, 2 * D), jnp.float32),
        pltpu.SemaphoreType.DMA,
        pltpu.SemaphoreType.DMA,
    ],
)
def _sc_pool(xw_hbm, xb_hbm, table_hbm, out_hbm, idxw, idxb, rw, rb, ostg,
             sem_w, sem_b):
    wid = lax.axis_index("s") * 2 + lax.axis_index("c")
    base = wid * BPW

    def group_body(g, _):
        pltpu.sync_copy(xw_hbm.at[pl.ds(base + g * GRP, GRP)], idxw)
        pltpu.sync_copy(xb_hbm.at[pl.ds(base + g * GRP, GRP)], idxb)

        def row_body(i, _):
            cpw = pltpu.async_copy(table_hbm.at[idxw.at[i]], rw, sem_w)
            cpb = pltpu.async_copy(table_hbm.at[idxb.at[i]], rb, sem_b)
            cpw.wait()
            _accum_into(rw, ostg, i, 0)
            cpb.wait()
            _accum_into(rb, ostg, i, D)
            return 0
        lax.fori_loop(0, GRP, row_body, 0)
        pltpu.sync_copy(ostg, out_hbm.at[pl.ds(base + g * GRP, GRP)])
        return 0

    lax.fori_loop(0, BPW // GRP, group_body, 0)


def _mlp_body(x_ref, w2_ref, b2_ref, w3_ref, b3_ref, w4_ref, b4_ref, o_ref):
    x = jnp.maximum(x_ref[:], 0.0)
    h = jnp.dot(x, w2_ref[:], preferred_element_type=jnp.float32) + b2_ref[:]
    h = jnp.maximum(h, 0.0)
    h = jnp.dot(h, w3_ref[:], preferred_element_type=jnp.float32) + b3_ref[:]
    h = jnp.maximum(h, 0.0)
    o_ref[:] = jnp.dot(h, w4_ref[:], preferred_element_type=jnp.float32) + b4_ref[:]


def _mlp(pooled, W2, b2, W3, b3, W4p, b4p):
    blk = 512
    return pl.pallas_call(
        _mlp_body,
        grid=(B // blk,),
        in_specs=[
            pl.BlockSpec((blk, 2 * D), lambda i: (i, 0)),
            pl.BlockSpec((2 * D, 32), lambda i: (0, 0)),
            pl.BlockSpec((1, 32), lambda i: (0, 0)),
            pl.BlockSpec((32, 32), lambda i: (0, 0)),
            pl.BlockSpec((1, 32), lambda i: (0, 0)),
            pl.BlockSpec((32, 128), lambda i: (0, 0)),
            pl.BlockSpec((1, 128), lambda i: (0, 0)),
        ],
        out_specs=pl.BlockSpec((blk, 128), lambda i: (i, 0)),
        out_shape=jax.ShapeDtypeStruct((B, 128), jnp.float32),
    )(pooled, W2, b2, W3, b3, W4p, b4p)


def kernel(x_w, x_b, table, W2, b2, W3, b3, W4, b4):
    pooled = _sc_pool(x_w.astype(jnp.int32), x_b.astype(jnp.int32), table)
    W4p = jnp.pad(W4, ((0, 0), (0, 127)))
    b4p = jnp.pad(b4.reshape(1, 1), ((0, 0), (0, 127)))
    out = _mlp(pooled, W2, b2.reshape(1, 32), W3, b3.reshape(1, 32), W4p, b4p)
    return out[:, :1]


# speedup vs baseline: 16.9787x; 2.0870x over previous
import functools

import jax
import jax.numpy as jnp
from jax import lax
from jax.experimental import pallas as pl
from jax.experimental.pallas import tpu as pltpu
from jax.experimental.pallas import tpu_sc as plsc

B = 16384
L = 50
D = 128
NW = 32
BPW = B // NW
GRP = 32
VL = 16


NBUF = 4
UNR = 5


def _accum_into(rows, stg, j, col0):
    def body(l5, accs):
        for u in range(UNR):
            accs = tuple(accs[c] + rows[l5 * UNR + u, pl.ds(c * VL, VL)]
                         for c in range(D // VL))
        return accs
    zero = jnp.zeros((VL,), jnp.float32)
    accs = lax.fori_loop(0, L // UNR, body, (zero,) * (D // VL))
    for c in range(D // VL):
        stg[j, pl.ds(col0 + c * VL, VL)] = accs[c]


_sc_mesh = plsc.VectorSubcoreMesh(core_axis_name="c", subcore_axis_name="s")


@functools.partial(
    pl.kernel,
    out_type=jax.ShapeDtypeStruct((B, 2 * D), jnp.float32),
    mesh=_sc_mesh,
    scratch_types=[
        pltpu.VMEM((GRP, L), jnp.int32),
        pltpu.VMEM((GRP, L), jnp.int32),
        pltpu.VMEM((NBUF, L, D), jnp.float32),
        pltpu.VMEM((NBUF, L, D), jnp.float32),
        pltpu.VMEM((GRP, 2 * D), jnp.float32),
        [pltpu.SemaphoreType.DMA] * NBUF,
        [pltpu.SemaphoreType.DMA] * NBUF,
    ],
)
def _sc_pool(xw_hbm, xb_hbm, table_hbm, out_hbm, idxw, idxb, rw, rb, ostg,
             sems_w, sems_b):
    wid = lax.axis_index("s") * 2 + lax.axis_index("c")
    base = wid * BPW

    def issue(r, u):
        pltpu.async_copy(table_hbm.at[idxw.at[r]], rw.at[u], sems_w[u])
        pltpu.async_copy(table_hbm.at[idxb.at[r]], rb.at[u], sems_b[u])

    def wait_and_acc(r, u):
        dummy = table_hbm.at[idxw.at[r]]
        pltpu.make_async_copy(dummy, rw.at[u], sems_w[u]).wait()
        _accum_into(rw.at[u], ostg, r, 0)
        pltpu.make_async_copy(dummy, rb.at[u], sems_b[u]).wait()
        _accum_into(rb.at[u], ostg, r, D)

    def group_body(g, _):
        pltpu.sync_copy(xw_hbm.at[pl.ds(base + g * GRP, GRP)], idxw)
        pltpu.sync_copy(xb_hbm.at[pl.ds(base + g * GRP, GRP)], idxb)
        for a in range(NBUF - 1):
            issue(a, a)
        def q_body(q, _):
            r = q * NBUF
            for u in range(NBUF):
                issue(r + u + NBUF - 1, (u + NBUF - 1) % NBUF)
                wait_and_acc(r + u, u)
            return 0
        lax.fori_loop(0, GRP // NBUF - 1, q_body, 0)
        issue(GRP - 1, (GRP - 1) % NBUF)
        for u in range(NBUF):
            wait_and_acc(GRP - NBUF + u, u)
        pltpu.sync_copy(ostg, out_hbm.at[pl.ds(base + g * GRP, GRP)])
        return 0

    lax.fori_loop(0, BPW // GRP, group_body, 0)


def _mlp_body(x_ref, w2_ref, b2_ref, w3_ref, b3_ref, w4_ref, b4_ref, o_ref):
    x = jnp.maximum(x_ref[:], 0.0)
    h = jnp.dot(x, w2_ref[:], preferred_element_type=jnp.float32) + b2_ref[:]
    h = jnp.maximum(h, 0.0)
    h = jnp.dot(h, w3_ref[:], preferred_element_type=jnp.float32) + b3_ref[:]
    h = jnp.maximum(h, 0.0)
    o_ref[:] = jnp.dot(h, w4_ref[:], preferred_element_type=jnp.float32) + b4_ref[:]


def _mlp(pooled, W2, b2, W3, b3, W4p, b4p):
    blk = 512
    return pl.pallas_call(
        _mlp_body,
        grid=(B // blk,),
        in_specs=[
            pl.BlockSpec((blk, 2 * D), lambda i: (i, 0)),
            pl.BlockSpec((2 * D, 32), lambda i: (0, 0)),
            pl.BlockSpec((1, 32), lambda i: (0, 0)),
            pl.BlockSpec((32, 32), lambda i: (0, 0)),
            pl.BlockSpec((1, 32), lambda i: (0, 0)),
            pl.BlockSpec((32, 128), lambda i: (0, 0)),
            pl.BlockSpec((1, 128), lambda i: (0, 0)),
        ],
        out_specs=pl.BlockSpec((blk, 128), lambda i: (i, 0)),
        out_shape=jax.ShapeDtypeStruct((B, 128), jnp.float32),
    )(pooled, W2, b2, W3, b3, W4p, b4p)


def kernel(x_w, x_b, table, W2, b2, W3, b3, W4, b4):
    pooled = _sc_pool(x_w.astype(jnp.int32), x_b.astype(jnp.int32), table)
    W4p = jnp.pad(W4, ((0, 0), (0, 127)))
    b4p = jnp.pad(b4.reshape(1, 1), ((0, 0), (0, 127)))
    out = _mlp(pooled, W2, b2.reshape(1, 32), W3, b3.reshape(1, 32), W4p, b4p)
    return out[:, :1]
